# Initial kernel scaffold; baseline (speedup 1.0000x reference)
import jax
import jax.numpy as jnp
from jax.experimental import pallas as pl


def kernel(x, w_qv, w_dw, w_proj):
    raise NotImplementedError("write your pallas kernel here")


# speedup vs baseline: 1.4983x; 1.4983x over previous
import math

import jax
import jax.numpy as jnp
from jax.experimental import pallas as pl
from jax.experimental.pallas import tpu as pltpu


def _attn_body(x_ref, wp_ref, m_ref, g_ref, *, temp, ch):
    j = pl.program_id(1)
    ns = pl.num_programs(1)

    @pl.when(j == 0)
    def _init():
        g_ref[...] = jnp.zeros_like(g_ref)

    xc = x_ref[0]
    g_ref[...] += jax.lax.dot_general(
        xc, xc, (((1,), (1,)), ((), ())),
        preferred_element_type=jnp.float32)

    @pl.when(j == ns - 1)
    def _finish():
        c = g_ref.shape[0]
        g = g_ref[...] * temp
        row = jax.lax.broadcasted_iota(jnp.int32, (c, c), 0) // ch
        col = jax.lax.broadcasted_iota(jnp.int32, (c, c), 1) // ch
        g = jnp.where(row == col, g, -1e30)
        g = g - jnp.max(g, axis=1, keepdims=True)
        e = jnp.exp(g)
        a = e / jnp.sum(e, axis=1, keepdims=True)
        m_ref[0] = jnp.dot(wp_ref[...], a, preferred_element_type=jnp.float32)


def _main_body(m_ref, wq_ref, wdw_ref, xm_ref, xt_ref, xb_ref, y_ref,
               xvp_ref, *, rows, w):
    i = pl.program_id(1)
    nr = pl.num_programs(1)
    sc = rows * w
    c = wq_ref.shape[0]

    wq = wq_ref[...]
    xvp_ref[:, w:w + sc] = jnp.dot(wq, xm_ref[0],
                                   preferred_element_type=jnp.float32)
    top = jnp.dot(wq, xt_ref[0], preferred_element_type=jnp.float32)
    xvp_ref[:, 0:w] = jnp.where(i == 0, 0.0, top)
    bot = jnp.dot(wq, xb_ref[0], preferred_element_type=jnp.float32)
    xvp_ref[:, w + sc:2 * w + sc] = jnp.where(i == nr - 1, 0.0, bot)

    lane = jax.lax.broadcasted_iota(jnp.int32, (c, sc), 1) % w
    acc = jnp.zeros((c, sc), jnp.float32)
    for di in (0, 1, 2):
        base = xvp_ref[:, di * w:di * w + sc]
        for dj in (0, 1, 2):
            wk = wdw_ref[:, di * 3 + dj:di * 3 + dj + 1]
            if dj == 0:
                sh = jnp.where(lane == 0, 0.0, jnp.roll(base, 1, axis=1))
            elif dj == 1:
                sh = base
            else:
                sh = jnp.where(lane == w - 1, 0.0, jnp.roll(base, -1, axis=1))
            acc = acc + wk * sh
    y_ref[0] = jnp.dot(m_ref[0], acc, preferred_element_type=jnp.float32)


def kernel(x, w_qv, w_dw, w_proj):
    b, c, h, w = x.shape
    s = h * w
    heads = 8
    ch = c // heads
    temp = 1.0 / math.sqrt(c)

    x2 = x.reshape(b, c, s)
    w_dw9 = w_dw.reshape(c, 9)

    sc_a = 2048 if s % 2048 == 0 else s
    ns = s // sc_a
    m = pl.pallas_call(
        lambda *refs: _attn_body(*refs, temp=temp, ch=ch),
        out_shape=jax.ShapeDtypeStruct((b, c, c), jnp.float32),
        grid=(b, ns),
        in_specs=[
            pl.BlockSpec((1, c, sc_a), lambda bi, j: (bi, 0, j)),
            pl.BlockSpec((c, c), lambda bi, j: (0, 0)),
        ],
        out_specs=pl.BlockSpec((1, c, c), lambda bi, j: (bi, 0, 0)),
        scratch_shapes=[pltpu.VMEM((c, c), jnp.float32)],
        compiler_params=pltpu.CompilerParams(
            dimension_semantics=("parallel", "arbitrary"),
            vmem_limit_bytes=56 * 1024 * 1024,
        ),
        name="sga_attn",
    )(x2, w_proj)

    rows = 32 if h % 32 == 0 else h
    nr = h // rows
    sc = rows * w

    y2 = pl.pallas_call(
        lambda *refs: _main_body(*refs, rows=rows, w=w),
        out_shape=jax.ShapeDtypeStruct((b, c, s), jnp.float32),
        grid=(b, nr),
        in_specs=[
            pl.BlockSpec((1, c, c), lambda bi, i: (bi, 0, 0)),
            pl.BlockSpec((c, c), lambda bi, i: (0, 0)),
            pl.BlockSpec((c, 9), lambda bi, i: (0, 0)),
            pl.BlockSpec((1, c, sc), lambda bi, i: (bi, 0, i)),
            pl.BlockSpec((1, c, w),
                         lambda bi, i: (bi, 0, jnp.maximum(i * rows - 1, 0))),
            pl.BlockSpec((1, c, w),
                         lambda bi, i: (bi, 0, jnp.minimum(i * rows + rows,
                                                           h - 1))),
        ],
        out_specs=pl.BlockSpec((1, c, sc), lambda bi, i: (bi, 0, i)),
        scratch_shapes=[pltpu.VMEM((c, sc + 2 * w), jnp.float32)],
        compiler_params=pltpu.CompilerParams(
            dimension_semantics=("parallel", "arbitrary"),
            vmem_limit_bytes=56 * 1024 * 1024,
        ),
        name="sga_main",
    )(m, w_qv, w_dw9, x2, x2, x2)

    return y2.reshape(b, c, h, w)


# speedup vs baseline: 2.2091x; 1.4744x over previous
import math

import jax
import jax.numpy as jnp
from jax.experimental import pallas as pl
from jax.experimental.pallas import tpu as pltpu


def _attn_body(x_ref, wp_ref, m_ref, g_ref, *, temp, ch):
    j = pl.program_id(1)
    ns = pl.num_programs(1)

    @pl.when(j == 0)
    def _init():
        g_ref[...] = jnp.zeros_like(g_ref)

    blk = x_ref.shape
    xc = x_ref[0].reshape(blk[1], blk[2] * blk[3])
    g_ref[...] += jax.lax.dot_general(
        xc, xc, (((1,), (1,)), ((), ())),
        preferred_element_type=jnp.float32)

    @pl.when(j == ns - 1)
    def _finish():
        c = g_ref.shape[0]
        g = g_ref[...] * temp
        row = jax.lax.broadcasted_iota(jnp.int32, (c, c), 0) // ch
        col = jax.lax.broadcasted_iota(jnp.int32, (c, c), 1) // ch
        g = jnp.where(row == col, g, -1e30)
        g = g - jnp.max(g, axis=1, keepdims=True)
        e = jnp.exp(g)
        a = e / jnp.sum(e, axis=1, keepdims=True)
        m_ref[0] = jnp.dot(wp_ref[...], a, preferred_element_type=jnp.float32)


def _main_body(m_ref, wq_ref, wdw_ref, xm_ref, xt_ref, xb_ref, y_ref,
               xvp_ref, *, rows, w):
    i = pl.program_id(1)
    nr = pl.num_programs(1)
    sc = rows * w
    c = wq_ref.shape[0]

    wq = wq_ref[...]
    xm = xm_ref[0].reshape(c, sc)
    xvp_ref[:, w:w + sc] = jnp.dot(wq, xm,
                                   preferred_element_type=jnp.float32)
    xt = xt_ref[0, :, 7:8, :].reshape(c, w)
    top = jnp.dot(wq, xt, preferred_element_type=jnp.float32)
    xvp_ref[:, 0:w] = jnp.where(i == 0, 0.0, top)
    xb = xb_ref[0, :, 0:1, :].reshape(c, w)
    bot = jnp.dot(wq, xb, preferred_element_type=jnp.float32)
    xvp_ref[:, w + sc:2 * w + sc] = jnp.where(i == nr - 1, 0.0, bot)

    lane = jax.lax.broadcasted_iota(jnp.int32, (c, sc), 1) % w
    acc = jnp.zeros((c, sc), jnp.float32)
    for di in (0, 1, 2):
        base = xvp_ref[:, di * w:di * w + sc]
        for dj in (0, 1, 2):
            wk = wdw_ref[:, di * 3 + dj:di * 3 + dj + 1]
            if dj == 0:
                sh = jnp.where(lane == 0, 0.0, jnp.roll(base, 1, axis=1))
            elif dj == 1:
                sh = base
            else:
                sh = jnp.where(lane == w - 1, 0.0, jnp.roll(base, -1, axis=1))
            acc = acc + wk * sh
    y = jnp.dot(m_ref[0], acc, preferred_element_type=jnp.float32)
    y_ref[0] = y.reshape(c, rows, w)


def kernel(x, w_qv, w_dw, w_proj):
    b, c, h, w = x.shape
    s = h * w
    heads = 8
    ch = c // heads
    temp = 1.0 / math.sqrt(c)

    w_dw9 = w_dw.reshape(c, 9)

    rows_a = 16 if h % 16 == 0 else h
    ns = h // rows_a
    m = pl.pallas_call(
        lambda *refs: _attn_body(*refs, temp=temp, ch=ch),
        out_shape=jax.ShapeDtypeStruct((b, c, c), jnp.float32),
        grid=(b, ns),
        in_specs=[
            pl.BlockSpec((1, c, rows_a, w), lambda bi, j: (bi, 0, j, 0)),
            pl.BlockSpec((c, c), lambda bi, j: (0, 0)),
        ],
        out_specs=pl.BlockSpec((1, c, c), lambda bi, j: (bi, 0, 0)),
        scratch_shapes=[pltpu.VMEM((c, c), jnp.float32)],
        compiler_params=pltpu.CompilerParams(
            dimension_semantics=("parallel", "arbitrary"),
            vmem_limit_bytes=56 * 1024 * 1024,
        ),
        name="sga_attn",
    )(x, w_proj)

    rows = 16 if h % 16 == 0 else h
    nr = h // rows
    sc = rows * w

    y = pl.pallas_call(
        lambda *refs: _main_body(*refs, rows=rows, w=w),
        out_shape=jax.ShapeDtypeStruct((b, c, h, w), jnp.float32),
        grid=(b, nr),
        in_specs=[
            pl.BlockSpec((1, c, c), lambda bi, i: (bi, 0, 0)),
            pl.BlockSpec((c, c), lambda bi, i: (0, 0)),
            pl.BlockSpec((c, 9), lambda bi, i: (0, 0)),
            pl.BlockSpec((1, c, rows, w), lambda bi, i: (bi, 0, i, 0)),
            pl.BlockSpec((1, c, 8, w),
                         lambda bi, i: (bi, 0,
                                        jnp.maximum(i * (rows // 8) - 1, 0),
                                        0)),
            pl.BlockSpec((1, c, 8, w),
                         lambda bi, i: (bi, 0,
                                        jnp.minimum(i * (rows // 8) +
                                                    rows // 8,
                                                    h // 8 - 1), 0)),
        ],
        out_specs=pl.BlockSpec((1, c, rows, w), lambda bi, i: (bi, 0, i, 0)),
        scratch_shapes=[pltpu.VMEM((c, sc + 2 * w), jnp.float32)],
        compiler_params=pltpu.CompilerParams(
            dimension_semantics=("parallel", "arbitrary"),
            vmem_limit_bytes=56 * 1024 * 1024,
        ),
        name="sga_main",
    )(m, w_qv, w_dw9, x, x, x)

    return y


# speedup vs baseline: 3.3193x; 1.5026x over previous
import math

import jax
import jax.numpy as jnp
from jax.experimental import pallas as pl
from jax.experimental.pallas import tpu as pltpu


def _attn_body(x_ref, wp_ref, m_ref, g_ref, *, temp, ch):
    j = pl.program_id(2)
    ns = pl.num_programs(2)

    @pl.when(j == 0)
    def _init():
        g_ref[...] = jnp.zeros_like(g_ref)

    blk = x_ref.shape
    xc = x_ref[0].reshape(blk[1], blk[2] * blk[3])
    g_ref[...] += jax.lax.dot_general(
        xc, xc, (((1,), (1,)), ((), ())),
        preferred_element_type=jnp.float32)

    @pl.when(j == ns - 1)
    def _finish():
        c = g_ref.shape[0]
        g = g_ref[...] * temp
        row = jax.lax.broadcasted_iota(jnp.int32, (c, c), 0) // ch
        col = jax.lax.broadcasted_iota(jnp.int32, (c, c), 1) // ch
        g = jnp.where(row == col, g, -1e30)
        g = g - jnp.max(g, axis=1, keepdims=True)
        e = jnp.exp(g)
        a = e / jnp.sum(e, axis=1, keepdims=True)
        m_ref[0] = jnp.dot(wp_ref[...], a, preferred_element_type=jnp.float32)


def _main_body(m_ref, wq_ref, wdw_ref, xm_ref, xt_ref, xb_ref, y_ref,
               xvp_ref, *, rows, w):
    i = pl.program_id(2)
    nr = pl.num_programs(2)
    sc = rows * w
    c = wq_ref.shape[0]

    wq = wq_ref[...]
    xm = xm_ref[0].reshape(c, sc)
    xvp_ref[:, w:w + sc] = jnp.dot(wq, xm,
                                   preferred_element_type=jnp.float32)
    xt = xt_ref[0, :, 7:8, :].reshape(c, w)
    top = jnp.dot(wq, xt, preferred_element_type=jnp.float32)
    xvp_ref[:, 0:w] = jnp.where(i == 0, 0.0, top)
    xb = xb_ref[0, :, 0:1, :].reshape(c, w)
    bot = jnp.dot(wq, xb, preferred_element_type=jnp.float32)
    xvp_ref[:, w + sc:2 * w + sc] = jnp.where(i == nr - 1, 0.0, bot)

    lane = jax.lax.broadcasted_iota(jnp.int32, (c, sc), 1) % w

    def hmix(dj):
        r = wdw_ref[:, dj:dj + 1] * xvp_ref[:, 0:sc]
        r = r + wdw_ref[:, 3 + dj:4 + dj] * xvp_ref[:, w:w + sc]
        r = r + wdw_ref[:, 6 + dj:7 + dj] * xvp_ref[:, 2 * w:2 * w + sc]
        return r

    acc = hmix(1)
    acc = acc + jnp.where(lane == 0, 0.0, jnp.roll(hmix(0), 1, axis=1))
    acc = acc + jnp.where(lane == w - 1, 0.0,
                          jnp.roll(hmix(2), -1, axis=1))
    y = jnp.dot(m_ref[0], acc, preferred_element_type=jnp.float32)
    y_ref[0] = y.reshape(c, rows, w)


def kernel(x, w_qv, w_dw, w_proj):
    b, c, h, w = x.shape
    s = h * w
    heads = 8
    ch = c // heads
    temp = 1.0 / math.sqrt(c)

    w_dw9 = w_dw.reshape(c, 9)

    rows_a = 16 if h % 16 == 0 else h
    ns = h // rows_a
    nc = 1
    bc = b // nc
    m = pl.pallas_call(
        lambda *refs: _attn_body(*refs, temp=temp, ch=ch),
        out_shape=jax.ShapeDtypeStruct((b, c, c), jnp.float32),
        grid=(nc, bc, ns),
        in_specs=[
            pl.BlockSpec((1, c, rows_a, w),
                         lambda ci, bi, j: (ci * bc + bi, 0, j, 0)),
            pl.BlockSpec((c, c), lambda ci, bi, j: (0, 0)),
        ],
        out_specs=pl.BlockSpec((1, c, c),
                               lambda ci, bi, j: (ci * bc + bi, 0, 0)),
        scratch_shapes=[pltpu.VMEM((c, c), jnp.float32)],
        compiler_params=pltpu.CompilerParams(
            dimension_semantics=("parallel", "parallel", "arbitrary"),
            vmem_limit_bytes=56 * 1024 * 1024,
        ),
        name="sga_attn",
    )(x, w_proj)

    rows = 16 if h % 16 == 0 else h
    nr = h // rows
    sc = rows * w

    y = pl.pallas_call(
        lambda *refs: _main_body(*refs, rows=rows, w=w),
        out_shape=jax.ShapeDtypeStruct((b, c, h, w), jnp.float32),
        grid=(nc, bc, nr),
        in_specs=[
            pl.BlockSpec((1, c, c), lambda ci, bi, i: (ci * bc + bi, 0, 0)),
            pl.BlockSpec((c, c), lambda ci, bi, i: (0, 0)),
            pl.BlockSpec((c, 9), lambda ci, bi, i: (0, 0)),
            pl.BlockSpec((1, c, rows, w),
                         lambda ci, bi, i: (ci * bc + bi, 0, i, 0)),
            pl.BlockSpec((1, c, 8, w),
                         lambda ci, bi, i: (ci * bc + bi, 0,
                                            jnp.maximum(
                                                i * (rows // 8) - 1, 0),
                                            0)),
            pl.BlockSpec((1, c, 8, w),
                         lambda ci, bi, i: (ci * bc + bi, 0,
                                            jnp.minimum(
                                                i * (rows // 8) + rows // 8,
                                                h // 8 - 1), 0)),
        ],
        out_specs=pl.BlockSpec((1, c, rows, w),
                               lambda ci, bi, i: (ci * bc + bi, 0, i, 0)),
        scratch_shapes=[pltpu.VMEM((c, sc + 2 * w), jnp.float32)],
        compiler_params=pltpu.CompilerParams(
            dimension_semantics=("parallel", "parallel", "arbitrary"),
            vmem_limit_bytes=56 * 1024 * 1024,
        ),
        name="sga_main",
    )(m, w_qv, w_dw9, x, x, x)

    return y


# speedup vs baseline: 3.3411x; 1.0066x over previous
import math

import jax
import jax.numpy as jnp
from jax.experimental import pallas as pl
from jax.experimental.pallas import tpu as pltpu


def _attn_body(x_ref, wp_ref, m_ref, g_ref, *, temp, ch):
    j = pl.program_id(2)
    ns = pl.num_programs(2)

    @pl.when(j == 0)
    def _init():
        g_ref[...] = jnp.zeros_like(g_ref)

    blk = x_ref.shape
    xc = x_ref[0].reshape(blk[1], blk[2] * blk[3]).astype(jnp.bfloat16)
    g_ref[...] += jax.lax.dot_general(
        xc, xc, (((1,), (1,)), ((), ())),
        preferred_element_type=jnp.float32)

    @pl.when(j == ns - 1)
    def _finish():
        c = g_ref.shape[0]
        g = g_ref[...] * temp
        row = jax.lax.broadcasted_iota(jnp.int32, (c, c), 0) // ch
        col = jax.lax.broadcasted_iota(jnp.int32, (c, c), 1) // ch
        g = jnp.where(row == col, g, -1e30)
        g = g - jnp.max(g, axis=1, keepdims=True)
        e = jnp.exp(g)
        a = e / jnp.sum(e, axis=1, keepdims=True)
        m_ref[0] = jnp.dot(wp_ref[...], a, preferred_element_type=jnp.float32)


def _main_body(m_ref, wq_ref, wdw_ref, xm_ref, xt_ref, xb_ref, y_ref,
               xvp_ref, *, rows, w):
    i = pl.program_id(2)
    nr = pl.num_programs(2)
    sc = rows * w
    c = wq_ref.shape[0]

    wq = wq_ref[...]
    xm = xm_ref[0].reshape(c, sc)
    xvp_ref[:, w:w + sc] = jnp.dot(wq, xm,
                                   preferred_element_type=jnp.float32)
    xt = xt_ref[0, :, 7:8, :].reshape(c, w)
    top = jnp.dot(wq, xt, preferred_element_type=jnp.float32)
    xvp_ref[:, 0:w] = jnp.where(i == 0, 0.0, top)
    xb = xb_ref[0, :, 0:1, :].reshape(c, w)
    bot = jnp.dot(wq, xb, preferred_element_type=jnp.float32)
    xvp_ref[:, w + sc:2 * w + sc] = jnp.where(i == nr - 1, 0.0, bot)

    lane = jax.lax.broadcasted_iota(jnp.int32, (c, sc), 1) % w

    def hmix(dj):
        r = wdw_ref[:, dj:dj + 1] * xvp_ref[:, 0:sc]
        r = r + wdw_ref[:, 3 + dj:4 + dj] * xvp_ref[:, w:w + sc]
        r = r + wdw_ref[:, 6 + dj:7 + dj] * xvp_ref[:, 2 * w:2 * w + sc]
        return r

    acc = hmix(1)
    acc = acc + jnp.where(lane == 0, 0.0, jnp.roll(hmix(0), 1, axis=1))
    acc = acc + jnp.where(lane == w - 1, 0.0,
                          jnp.roll(hmix(2), -1, axis=1))
    y = jnp.dot(m_ref[0], acc, preferred_element_type=jnp.float32)
    y_ref[0] = y.reshape(c, rows, w)


def kernel(x, w_qv, w_dw, w_proj):
    b, c, h, w = x.shape
    s = h * w
    heads = 8
    ch = c // heads
    temp = 1.0 / math.sqrt(c)

    w_dw9 = w_dw.reshape(c, 9)

    rows_a = 32 if h % 32 == 0 else h
    ns = h // rows_a
    nc = 1
    bc = b // nc
    m = pl.pallas_call(
        lambda *refs: _attn_body(*refs, temp=temp, ch=ch),
        out_shape=jax.ShapeDtypeStruct((b, c, c), jnp.float32),
        grid=(nc, bc, ns),
        in_specs=[
            pl.BlockSpec((1, c, rows_a, w),
                         lambda ci, bi, j: (ci * bc + bi, 0, j, 0)),
            pl.BlockSpec((c, c), lambda ci, bi, j: (0, 0)),
        ],
        out_specs=pl.BlockSpec((1, c, c),
                               lambda ci, bi, j: (ci * bc + bi, 0, 0)),
        scratch_shapes=[pltpu.VMEM((c, c), jnp.float32)],
        compiler_params=pltpu.CompilerParams(
            dimension_semantics=("parallel", "parallel", "arbitrary"),
            vmem_limit_bytes=56 * 1024 * 1024,
        ),
        name="sga_attn",
    )(x, w_proj)

    rows = 32 if h % 32 == 0 else h
    nr = h // rows
    sc = rows * w

    y = pl.pallas_call(
        lambda *refs: _main_body(*refs, rows=rows, w=w),
        out_shape=jax.ShapeDtypeStruct((b, c, h, w), jnp.float32),
        grid=(nc, bc, nr),
        in_specs=[
            pl.BlockSpec((1, c, c), lambda ci, bi, i: (ci * bc + bi, 0, 0)),
            pl.BlockSpec((c, c), lambda ci, bi, i: (0, 0)),
            pl.BlockSpec((c, 9), lambda ci, bi, i: (0, 0)),
            pl.BlockSpec((1, c, rows, w),
                         lambda ci, bi, i: (ci * bc + bi, 0, i, 0)),
            pl.BlockSpec((1, c, 8, w),
                         lambda ci, bi, i: (ci * bc + bi, 0,
                                            jnp.maximum(
                                                i * (rows // 8) - 1, 0),
                                            0)),
            pl.BlockSpec((1, c, 8, w),
                         lambda ci, bi, i: (ci * bc + bi, 0,
                                            jnp.minimum(
                                                i * (rows // 8) + rows // 8,
                                                h // 8 - 1), 0)),
        ],
        out_specs=pl.BlockSpec((1, c, rows, w),
                               lambda ci, bi, i: (ci * bc + bi, 0, i, 0)),
        scratch_shapes=[pltpu.VMEM((c, sc + 2 * w), jnp.float32)],
        compiler_params=pltpu.CompilerParams(
            dimension_semantics=("parallel", "parallel", "arbitrary"),
            vmem_limit_bytes=56 * 1024 * 1024,
        ),
        name="sga_main",
    )(m, w_qv, w_dw9, x, x, x)

    return y
